# baseline (speedup 1.0000x reference)
import functools

import jax
import jax.numpy as jnp
from jax.experimental import pallas as pl
from jax.experimental.pallas import tpu as pltpu

_NC = 10000
_ND = 64
_NH = 8
_RNN = 8
_T = 64
_B = 512
_NC_PAD = 10240


def _bigru_head_body(ce_ref, dbf_ref, dbb_ref, kcf_ref, kcb_ref,
                     rf_ref, rb_ref, brf_ref, brb_ref,
                     wdf_ref, wdb_ref, dbias_ref,
                     out_ref, f_scr, b_scr):
    ce = ce_ref[...]
    af = jnp.dot(kcf_ref[...], ce, preferred_element_type=jnp.float32)
    ab = jnp.dot(kcb_ref[...], ce, preferred_element_type=jnp.float32)
    rf = rf_ref[...]
    rb = rb_ref[...]
    brf = brf_ref[...]
    brb = brb_ref[...]

    def scan_dir(a, db_ref, rT, br, scr_ref, reverse):
        def step(i, h):
            t = (_T - 1 - i) if reverse else i
            d = db_ref[pl.ds(t, 1), :, :]
            mx = a + d.reshape(d.shape[1], d.shape[2])
            mh = jnp.dot(rT, h, preferred_element_type=jnp.float32) + br
            xz, xr, xh = mx[0:8], mx[8:16], mx[16:24]
            hz, hr, hh = mh[0:8], mh[8:16], mh[16:24]
            z = jax.nn.sigmoid(xz + hz)
            r = jax.nn.sigmoid(xr + hr)
            hc = jnp.tanh(xh + r * hh)
            hn = z * h + (1.0 - z) * hc
            scr_ref[pl.ds(t, 1), :, :] = hn.reshape(1, hn.shape[0], hn.shape[1])
            return hn
        h0 = jnp.zeros((_RNN, ce.shape[1]), jnp.float32)
        jax.lax.fori_loop(0, _T, step, h0)

    scan_dir(af, dbf_ref, rf, brf, f_scr, False)
    scan_dir(ab, dbb_ref, rb, brb, b_scr, True)

    wdf = wdf_ref[...]
    wdb = wdb_ref[...]
    dbias = dbias_ref[...]

    def head(t, _):
        f = f_scr[pl.ds(t, 1), :, :]
        b = b_scr[pl.ds(t, 1), :, :]
        f = f.reshape(f.shape[1], f.shape[2])
        b = b.reshape(b.shape[1], b.shape[2])
        lg = (jnp.dot(wdf, f, preferred_element_type=jnp.float32)
              + jnp.dot(wdb, b, preferred_element_type=jnp.float32) + dbias)
        m = jnp.max(lg, axis=0, keepdims=True)
        e = jnp.exp(lg - m)
        s = jnp.sum(e, axis=0, keepdims=True)
        p = e / s
        out_ref[pl.ds(t, 1), :, :] = p.reshape(1, p.shape[0], p.shape[1])
        return 0
    jax.lax.fori_loop(0, _T, head, 0)


def _bigru_head(comp_emb, dev_emb, params):
    pf, pb = params['gru_fwd'], params['gru_bwd']
    ceT = jnp.zeros((_RNN, _NC_PAD), jnp.float32).at[:, :_NC].set(comp_emb.T)

    def dproj(p):
        d = dev_emb @ p['kernel'][_RNN:] + p['bias'][0]
        return jnp.broadcast_to(d[:, :, None], (_T, 3 * _RNN, _B))

    dbf, dbb = dproj(pf), dproj(pb)
    kcf = pf['kernel'][:_RNN].T
    kcb = pb['kernel'][:_RNN].T
    rf, rb = pf['rkernel'].T, pb['rkernel'].T
    brf = pf['bias'][1][:, None]
    brb = pb['bias'][1][:, None]
    wd = params['dense_W'].T
    wdf, wdb = wd[:, :_RNN], wd[:, _RNN:]
    dbias = params['dense_b'][:, None]

    grid = _NC_PAD // _B
    const = lambda shape: pl.BlockSpec(shape, lambda i: tuple(0 for _ in shape))
    out = pl.pallas_call(
        _bigru_head_body,
        grid=(grid,),
        in_specs=[
            pl.BlockSpec((_RNN, _B), lambda i: (0, i)),
            const((_T, 3 * _RNN, _B)),
            const((_T, 3 * _RNN, _B)),
            const((3 * _RNN, _RNN)),
            const((3 * _RNN, _RNN)),
            const((3 * _RNN, _RNN)),
            const((3 * _RNN, _RNN)),
            const((3 * _RNN, 1)),
            const((3 * _RNN, 1)),
            const((3, _RNN)),
            const((3, _RNN)),
            const((3, 1)),
        ],
        out_specs=pl.BlockSpec((_T, 3, _B), lambda i: (0, 0, i)),
        out_shape=jax.ShapeDtypeStruct((_T, 3, _NC_PAD), jnp.float32),
        scratch_shapes=[
            pltpu.VMEM((_T, _RNN, _B), jnp.float32),
            pltpu.VMEM((_T, _RNN, _B), jnp.float32),
        ],
    )(ceT, dbf, dbb, kcf, kcb, rf, rb, brf, brb, wdf, wdb, dbias)
    return out.transpose(2, 0, 1)[:_NC]


def _dev_gat_body(x_ref, src_ref, dst_ref, *refs):
    out_ref = refs[-1]
    wrefs = refs[:-1]
    E, N = 1024, _ND
    src = src_ref[...]
    dst = dst_ref[...]
    iota_e_n = jax.lax.broadcasted_iota(jnp.int32, (E, N), 1)
    s_src = (iota_e_n == src.reshape(E, 1)).astype(jnp.float32)
    s_dst = (iota_e_n == dst.reshape(E, 1)).astype(jnp.float32)
    iota_n_e = jax.lax.broadcasted_iota(jnp.int32, (N, E), 0)
    dmask = (iota_n_e == dst.reshape(1, E))
    s_dstT = dmask.astype(jnp.float32)

    x = x_ref[...]
    layer_cfg = [(8, True), (8, True), (8, True), (1, False)]
    for li, (heads, use_elu) in enumerate(layer_cfg):
        W = wrefs[3 * li][...]
        a_l = wrefs[3 * li + 1][...]
        a_r = wrefs[3 * li + 2][...]
        z = jnp.dot(x, W, preferred_element_type=jnp.float32)
        zr = z.reshape(N, heads, _NH)
        el = jnp.sum(zr * a_l[None, :, :], axis=-1)
        er = jnp.sum(zr * a_r[None, :, :], axis=-1)
        el_s = jnp.dot(s_src, el, preferred_element_type=jnp.float32)
        er_d = jnp.dot(s_dst, er, preferred_element_type=jnp.float32)
        e = el_s + er_d
        e = jnp.where(e >= 0, e, 0.2 * e)
        neg = jnp.float32(-jnp.inf)
        emax_cols = []
        for h in range(heads):
            vh = e[:, h].reshape(1, E)
            mh = jnp.where(dmask, vh, neg)
            emax_cols.append(jnp.max(mh, axis=1, keepdims=True))
        emax = jnp.concatenate(emax_cols, axis=1)
        emax = jnp.where(jnp.isfinite(emax), emax, 0.0)
        emax_d = jnp.dot(s_dst, emax, preferred_element_type=jnp.float32)
        ee = jnp.exp(e - emax_d)
        denom = jnp.dot(s_dstT, ee, preferred_element_type=jnp.float32)
        denom_d = jnp.dot(s_dst, denom, preferred_element_type=jnp.float32)
        alpha = ee / (denom_d + 1e-9)
        z_s = jnp.dot(s_src, z, preferred_element_type=jnp.float32)
        w = z_s.reshape(E, heads, _NH) * alpha[:, :, None]
        w = w.reshape(E, heads * _NH)
        out = jnp.dot(s_dstT, w, preferred_element_type=jnp.float32)
        if use_elu:
            out = jnp.where(out > 0, out, jnp.exp(jnp.minimum(out, 0.0)) - 1.0)
        x = out
    out_ref[...] = x


def _dev_gat(dev_feat, dev_ei, params):
    src = dev_ei[0].reshape(1, -1).astype(jnp.int32)
    dst = dev_ei[1].reshape(1, -1).astype(jnp.int32)
    args = [dev_feat, src, dst]
    for i in range(4):
        p = params['dev%d' % i]
        args += [p['W'], p['a_l'], p['a_r']]
    specs = [pl.BlockSpec(a.shape, functools.partial(lambda nd, *_: tuple(0 for _ in range(nd)), a.ndim))
             for a in args]
    return pl.pallas_call(
        lambda *r: _dev_gat_body(*r),
        grid=(1,),
        in_specs=specs,
        out_specs=pl.BlockSpec((_ND, _NH), lambda i: (0, 0)),
        out_shape=jax.ShapeDtypeStruct((_ND, _NH), jnp.float32),
    )(*args)


def _gat_conv(x, src, dst, p, heads, out_dim, use_elu):
    N = x.shape[0]
    z = (x @ p['W']).reshape(N, heads, out_dim)
    el = jnp.sum(z * p['a_l'][None, :, :], axis=-1)
    er = jnp.sum(z * p['a_r'][None, :, :], axis=-1)
    e = jax.nn.leaky_relu(el[src] + er[dst], 0.2)
    emax = jax.ops.segment_max(e, dst, num_segments=N)
    emax = jnp.where(jnp.isfinite(emax), emax, 0.0)
    ee = jnp.exp(e - emax[dst])
    denom = jax.ops.segment_sum(ee, dst, num_segments=N)
    alpha = ee / (denom[dst] + 1e-9)
    out = jax.ops.segment_sum(z[src] * alpha[..., None], dst, num_segments=N)
    if use_elu:
        out = jax.nn.elu(out)
    return out.reshape(N, heads * out_dim)


def kernel(computation_features, device_features, comp_edge_index, dev_edge_index, params):
    comp_cfg = [(8, True), (8, True), (8, True), (1, False)]
    x = computation_features
    for i, (h, act) in enumerate(comp_cfg):
        x = _gat_conv(x, comp_edge_index[0], comp_edge_index[1],
                      params['comp%d' % i], h, _NH, act)
    comp_emb = x
    dev_emb = _dev_gat(device_features, dev_edge_index, params)
    return _bigru_head(comp_emb, dev_emb, params)


# speedup vs baseline: 10.9569x; 10.9569x over previous
import functools

import jax
import jax.numpy as jnp
from jax.experimental import pallas as pl
from jax.experimental.pallas import tpu as pltpu

_NC = 10000
_ND = 64
_NH = 8
_RNN = 8
_T = 64
_B = 512
_NC_PAD = 10240


def _bigru_head_body(ce_ref, dbf_ref, dbb_ref, kcf_ref, kcb_ref,
                     rf_ref, rb_ref, brf_ref, brb_ref,
                     wdf_ref, wdb_ref, dbias_ref,
                     out_ref, f_scr, b_scr):
    ce = ce_ref[...]
    af = jnp.dot(kcf_ref[...], ce, preferred_element_type=jnp.float32)
    ab = jnp.dot(kcb_ref[...], ce, preferred_element_type=jnp.float32)
    rf = rf_ref[...]
    rb = rb_ref[...]
    brf = brf_ref[...]
    brb = brb_ref[...]

    def scan_dir(a, db_ref, rT, br, scr_ref, reverse):
        def step(i, h):
            t = (_T - 1 - i) if reverse else i
            d = db_ref[pl.ds(t, 1), :, :]
            mx = a + d.reshape(d.shape[1], d.shape[2])
            mh = jnp.dot(rT, h, preferred_element_type=jnp.float32) + br
            xz, xr, xh = mx[0:8], mx[8:16], mx[16:24]
            hz, hr, hh = mh[0:8], mh[8:16], mh[16:24]
            z = jax.nn.sigmoid(xz + hz)
            r = jax.nn.sigmoid(xr + hr)
            hc = jnp.tanh(xh + r * hh)
            hn = z * h + (1.0 - z) * hc
            scr_ref[pl.ds(t, 1), :, :] = hn.reshape(1, hn.shape[0], hn.shape[1])
            return hn
        h0 = jnp.zeros((_RNN, ce.shape[1]), jnp.float32)
        jax.lax.fori_loop(0, _T, step, h0)

    scan_dir(af, dbf_ref, rf, brf, f_scr, False)
    scan_dir(ab, dbb_ref, rb, brb, b_scr, True)

    wdf = wdf_ref[...]
    wdb = wdb_ref[...]
    dbias = dbias_ref[...]

    def head(t, _):
        f = f_scr[pl.ds(t, 1), :, :]
        b = b_scr[pl.ds(t, 1), :, :]
        f = f.reshape(f.shape[1], f.shape[2])
        b = b.reshape(b.shape[1], b.shape[2])
        lg = (jnp.dot(wdf, f, preferred_element_type=jnp.float32)
              + jnp.dot(wdb, b, preferred_element_type=jnp.float32) + dbias)
        m = jnp.max(lg, axis=0, keepdims=True)
        e = jnp.exp(lg - m)
        s = jnp.sum(e, axis=0, keepdims=True)
        p = e / s
        out_ref[pl.ds(t, 1), :, :] = p.reshape(1, p.shape[0], p.shape[1])
        return 0
    jax.lax.fori_loop(0, _T, head, 0)


def _bigru_head(comp_emb, dev_emb, params):
    pf, pb = params['gru_fwd'], params['gru_bwd']
    ceT = jnp.zeros((_RNN, _NC_PAD), jnp.float32).at[:, :_NC].set(comp_emb.T)

    def dproj(p):
        d = dev_emb @ p['kernel'][_RNN:] + p['bias'][0]
        return jnp.broadcast_to(d[:, :, None], (_T, 3 * _RNN, _B))

    dbf, dbb = dproj(pf), dproj(pb)
    kcf = pf['kernel'][:_RNN].T
    kcb = pb['kernel'][:_RNN].T
    rf, rb = pf['rkernel'].T, pb['rkernel'].T
    brf = pf['bias'][1][:, None]
    brb = pb['bias'][1][:, None]
    wd = params['dense_W'].T
    wdf, wdb = wd[:, :_RNN], wd[:, _RNN:]
    dbias = params['dense_b'][:, None]

    grid = _NC_PAD // _B
    const = lambda shape: pl.BlockSpec(shape, lambda i: tuple(0 for _ in shape))
    out = pl.pallas_call(
        _bigru_head_body,
        grid=(grid,),
        in_specs=[
            pl.BlockSpec((_RNN, _B), lambda i: (0, i)),
            const((_T, 3 * _RNN, _B)),
            const((_T, 3 * _RNN, _B)),
            const((3 * _RNN, _RNN)),
            const((3 * _RNN, _RNN)),
            const((3 * _RNN, _RNN)),
            const((3 * _RNN, _RNN)),
            const((3 * _RNN, 1)),
            const((3 * _RNN, 1)),
            const((3, _RNN)),
            const((3, _RNN)),
            const((3, 1)),
        ],
        out_specs=pl.BlockSpec((_T, 3, _B), lambda i: (0, 0, i)),
        out_shape=jax.ShapeDtypeStruct((_T, 3, _NC_PAD), jnp.float32),
        scratch_shapes=[
            pltpu.VMEM((_T, _RNN, _B), jnp.float32),
            pltpu.VMEM((_T, _RNN, _B), jnp.float32),
        ],
    )(ceT, dbf, dbb, kcf, kcb, rf, rb, brf, brb, wdf, wdb, dbias)
    return out.transpose(2, 0, 1)[:_NC]


def _dev_gat_body(x_ref, src_ref, dst_ref, *refs):
    out_ref = refs[-1]
    wrefs = refs[:-1]
    E, N = 1024, _ND
    src = src_ref[...]
    dst = dst_ref[...]
    iota_e_n = jax.lax.broadcasted_iota(jnp.int32, (E, N), 1)
    s_src = (iota_e_n == src.reshape(E, 1)).astype(jnp.float32)
    s_dst = (iota_e_n == dst.reshape(E, 1)).astype(jnp.float32)
    iota_n_e = jax.lax.broadcasted_iota(jnp.int32, (N, E), 0)
    dmask = (iota_n_e == dst.reshape(1, E))
    s_dstT = dmask.astype(jnp.float32)

    x = x_ref[...]
    layer_cfg = [(8, True), (8, True), (8, True), (1, False)]
    for li, (heads, use_elu) in enumerate(layer_cfg):
        W = wrefs[3 * li][...]
        a_l = wrefs[3 * li + 1][...]
        a_r = wrefs[3 * li + 2][...]
        z = jnp.dot(x, W, preferred_element_type=jnp.float32)
        zr = z.reshape(N, heads, _NH)
        el = jnp.sum(zr * a_l[None, :, :], axis=-1)
        er = jnp.sum(zr * a_r[None, :, :], axis=-1)
        el_s = jnp.dot(s_src, el, preferred_element_type=jnp.float32)
        er_d = jnp.dot(s_dst, er, preferred_element_type=jnp.float32)
        e = el_s + er_d
        e = jnp.where(e >= 0, e, 0.2 * e)
        neg = jnp.float32(-jnp.inf)
        emax_cols = []
        for h in range(heads):
            vh = e[:, h].reshape(1, E)
            mh = jnp.where(dmask, vh, neg)
            emax_cols.append(jnp.max(mh, axis=1, keepdims=True))
        emax = jnp.concatenate(emax_cols, axis=1)
        emax = jnp.where(jnp.isfinite(emax), emax, 0.0)
        emax_d = jnp.dot(s_dst, emax, preferred_element_type=jnp.float32)
        ee = jnp.exp(e - emax_d)
        denom = jnp.dot(s_dstT, ee, preferred_element_type=jnp.float32)
        denom_d = jnp.dot(s_dst, denom, preferred_element_type=jnp.float32)
        alpha = ee / (denom_d + 1e-9)
        z_s = jnp.dot(s_src, z, preferred_element_type=jnp.float32)
        w = z_s.reshape(E, heads, _NH) * alpha[:, :, None]
        w = w.reshape(E, heads * _NH)
        out = jnp.dot(s_dstT, w, preferred_element_type=jnp.float32)
        if use_elu:
            out = jnp.where(out > 0, out, jnp.exp(jnp.minimum(out, 0.0)) - 1.0)
        x = out
    out_ref[...] = x


def _dev_gat(dev_feat, dev_ei, params):
    src = dev_ei[0].reshape(1, -1).astype(jnp.int32)
    dst = dev_ei[1].reshape(1, -1).astype(jnp.int32)
    args = [dev_feat, src, dst]
    for i in range(4):
        p = params['dev%d' % i]
        args += [p['W'], p['a_l'], p['a_r']]
    specs = [pl.BlockSpec(a.shape, functools.partial(lambda nd, *_: tuple(0 for _ in range(nd)), a.ndim))
             for a in args]
    return pl.pallas_call(
        lambda *r: _dev_gat_body(*r),
        grid=(1,),
        in_specs=specs,
        out_specs=pl.BlockSpec((_ND, _NH), lambda i: (0, 0)),
        out_shape=jax.ShapeDtypeStruct((_ND, _NH), jnp.float32),
    )(*args)


def _gat_conv(x, src, dst, p, heads, out_dim, use_elu):
    N = x.shape[0]
    z = x @ p['W']
    zr = z.reshape(N, heads, out_dim)
    el = jnp.sum(zr * p['a_l'][None, :, :], axis=-1)
    er = jnp.sum(zr * p['a_r'][None, :, :], axis=-1)
    src_tab = jnp.concatenate([z, el], axis=1)
    g_src = src_tab[src]
    er_d = er[dst]
    e = jax.nn.leaky_relu(g_src[:, heads * out_dim:] + er_d, 0.2)
    ee = jnp.exp(e - jnp.max(e))
    z_src = g_src[:, :heads * out_dim].reshape(-1, heads, out_dim)
    payload = jnp.concatenate(
        [ee, (z_src * ee[:, :, None]).reshape(-1, heads * out_dim)], axis=1)
    s = jax.ops.segment_sum(payload, dst, num_segments=N)
    denom = s[:, :heads]
    num = s[:, heads:].reshape(N, heads, out_dim)
    out = num / (denom[:, :, None] + 1e-9)
    if use_elu:
        out = jax.nn.elu(out)
    return out.reshape(N, heads * out_dim)


def kernel(computation_features, device_features, comp_edge_index, dev_edge_index, params):
    comp_cfg = [(8, True), (8, True), (8, True), (1, False)]
    x = computation_features
    for i, (h, act) in enumerate(comp_cfg):
        x = _gat_conv(x, comp_edge_index[0], comp_edge_index[1],
                      params['comp%d' % i], h, _NH, act)
    comp_emb = x
    dev_emb = _dev_gat(device_features, dev_edge_index, params)
    return _bigru_head(comp_emb, dev_emb, params)


# speedup vs baseline: 11.3019x; 1.0315x over previous
import functools

import jax
import jax.numpy as jnp
from jax.experimental import pallas as pl
from jax.experimental.pallas import tpu as pltpu

_NC = 10000
_ND = 64
_NH = 8
_RNN = 8
_T = 64
_B = 2048
_NC_PAD = 10240


def _bigru_head_body(ce_ref, dbf_ref, dbb_ref, kcf_ref, kcb_ref,
                     rf_ref, rb_ref, brf_ref, brb_ref,
                     wdf_ref, wdb_ref, dbias_ref,
                     out_ref, f_scr, b_scr):
    ce = ce_ref[...]
    af = jnp.dot(kcf_ref[...], ce, preferred_element_type=jnp.float32)
    ab = jnp.dot(kcb_ref[...], ce, preferred_element_type=jnp.float32)
    rf = rf_ref[...]
    rb = rb_ref[...]
    brf = brf_ref[...]
    brb = brb_ref[...]

    def scan_dir(a, db_ref, rT, br, scr_ref, reverse):
        def step(i, h):
            t = (_T - 1 - i) if reverse else i
            d = db_ref[pl.ds(t, 1), :, :]
            mx = a + d.reshape(d.shape[1], d.shape[2])
            mh = jnp.dot(rT, h, preferred_element_type=jnp.float32) + br
            xz, xr, xh = mx[0:8], mx[8:16], mx[16:24]
            hz, hr, hh = mh[0:8], mh[8:16], mh[16:24]
            z = jax.nn.sigmoid(xz + hz)
            r = jax.nn.sigmoid(xr + hr)
            hc = jnp.tanh(xh + r * hh)
            hn = z * h + (1.0 - z) * hc
            scr_ref[pl.ds(t, 1), :, :] = hn.reshape(1, hn.shape[0], hn.shape[1])
            return hn
        h0 = jnp.zeros((_RNN, ce.shape[1]), jnp.float32)
        jax.lax.fori_loop(0, _T, step, h0)

    scan_dir(af, dbf_ref, rf, brf, f_scr, False)
    scan_dir(ab, dbb_ref, rb, brb, b_scr, True)

    wdf = wdf_ref[...]
    wdb = wdb_ref[...]
    dbias = dbias_ref[...]

    def head(t, _):
        f = f_scr[pl.ds(t, 1), :, :]
        b = b_scr[pl.ds(t, 1), :, :]
        f = f.reshape(f.shape[1], f.shape[2])
        b = b.reshape(b.shape[1], b.shape[2])
        lg = (jnp.dot(wdf, f, preferred_element_type=jnp.float32)
              + jnp.dot(wdb, b, preferred_element_type=jnp.float32) + dbias)
        m = jnp.max(lg, axis=0, keepdims=True)
        e = jnp.exp(lg - m)
        s = jnp.sum(e, axis=0, keepdims=True)
        p = e / s
        out_ref[pl.ds(t, 1), :, :] = p.reshape(1, p.shape[0], p.shape[1])
        return 0
    jax.lax.fori_loop(0, _T, head, 0)


def _bigru_head(comp_emb, dev_emb, params):
    pf, pb = params['gru_fwd'], params['gru_bwd']
    ceT = jnp.zeros((_RNN, _NC_PAD), jnp.float32).at[:, :_NC].set(comp_emb.T)

    def dproj(p):
        d = dev_emb @ p['kernel'][_RNN:] + p['bias'][0]
        return jnp.broadcast_to(d[:, :, None], (_T, 3 * _RNN, _B))

    dbf, dbb = dproj(pf), dproj(pb)
    kcf = pf['kernel'][:_RNN].T
    kcb = pb['kernel'][:_RNN].T
    rf, rb = pf['rkernel'].T, pb['rkernel'].T
    brf = pf['bias'][1][:, None]
    brb = pb['bias'][1][:, None]
    wd = params['dense_W'].T
    wdf, wdb = wd[:, :_RNN], wd[:, _RNN:]
    dbias = params['dense_b'][:, None]

    grid = _NC_PAD // _B
    const = lambda shape: pl.BlockSpec(shape, lambda i: tuple(0 for _ in shape))
    out = pl.pallas_call(
        _bigru_head_body,
        grid=(grid,),
        in_specs=[
            pl.BlockSpec((_RNN, _B), lambda i: (0, i)),
            const((_T, 3 * _RNN, _B)),
            const((_T, 3 * _RNN, _B)),
            const((3 * _RNN, _RNN)),
            const((3 * _RNN, _RNN)),
            const((3 * _RNN, _RNN)),
            const((3 * _RNN, _RNN)),
            const((3 * _RNN, 1)),
            const((3 * _RNN, 1)),
            const((3, _RNN)),
            const((3, _RNN)),
            const((3, 1)),
        ],
        out_specs=pl.BlockSpec((_T, 3, _B), lambda i: (0, 0, i)),
        out_shape=jax.ShapeDtypeStruct((_T, 3, _NC_PAD), jnp.float32),
        scratch_shapes=[
            pltpu.VMEM((_T, _RNN, _B), jnp.float32),
            pltpu.VMEM((_T, _RNN, _B), jnp.float32),
        ],
    )(ceT, dbf, dbb, kcf, kcb, rf, rb, brf, brb, wdf, wdb, dbias)
    return out.transpose(2, 0, 1)[:_NC]


def _dev_gat_body(x_ref, src_ref, dst_ref, *refs):
    out_ref = refs[-1]
    wrefs = refs[:-1]
    E, N = 1024, _ND
    src = src_ref[...]
    dst = dst_ref[...]
    iota_e_n = jax.lax.broadcasted_iota(jnp.int32, (E, N), 1)
    s_src = (iota_e_n == src.reshape(E, 1)).astype(jnp.float32)
    s_dst = (iota_e_n == dst.reshape(E, 1)).astype(jnp.float32)
    iota_n_e = jax.lax.broadcasted_iota(jnp.int32, (N, E), 0)
    dmask = (iota_n_e == dst.reshape(1, E))
    s_dstT = dmask.astype(jnp.float32)

    x = x_ref[...]
    layer_cfg = [(8, True), (8, True), (8, True), (1, False)]
    for li, (heads, use_elu) in enumerate(layer_cfg):
        W = wrefs[3 * li][...]
        a_l = wrefs[3 * li + 1][...]
        a_r = wrefs[3 * li + 2][...]
        z = jnp.dot(x, W, preferred_element_type=jnp.float32)
        zr = z.reshape(N, heads, _NH)
        el = jnp.sum(zr * a_l[None, :, :], axis=-1)
        er = jnp.sum(zr * a_r[None, :, :], axis=-1)
        el_s = jnp.dot(s_src, el, preferred_element_type=jnp.float32)
        er_d = jnp.dot(s_dst, er, preferred_element_type=jnp.float32)
        e = el_s + er_d
        e = jnp.where(e >= 0, e, 0.2 * e)
        neg = jnp.float32(-jnp.inf)
        emax_cols = []
        for h in range(heads):
            vh = e[:, h].reshape(1, E)
            mh = jnp.where(dmask, vh, neg)
            emax_cols.append(jnp.max(mh, axis=1, keepdims=True))
        emax = jnp.concatenate(emax_cols, axis=1)
        emax = jnp.where(jnp.isfinite(emax), emax, 0.0)
        emax_d = jnp.dot(s_dst, emax, preferred_element_type=jnp.float32)
        ee = jnp.exp(e - emax_d)
        denom = jnp.dot(s_dstT, ee, preferred_element_type=jnp.float32)
        denom_d = jnp.dot(s_dst, denom, preferred_element_type=jnp.float32)
        alpha = ee / (denom_d + 1e-9)
        z_s = jnp.dot(s_src, z, preferred_element_type=jnp.float32)
        w = z_s.reshape(E, heads, _NH) * alpha[:, :, None]
        w = w.reshape(E, heads * _NH)
        out = jnp.dot(s_dstT, w, preferred_element_type=jnp.float32)
        if use_elu:
            out = jnp.where(out > 0, out, jnp.exp(jnp.minimum(out, 0.0)) - 1.0)
        x = out
    out_ref[...] = x


def _dev_gat(dev_feat, dev_ei, params):
    src = dev_ei[0].reshape(1, -1).astype(jnp.int32)
    dst = dev_ei[1].reshape(1, -1).astype(jnp.int32)
    args = [dev_feat, src, dst]
    for i in range(4):
        p = params['dev%d' % i]
        args += [p['W'], p['a_l'], p['a_r']]
    specs = [pl.BlockSpec(a.shape, functools.partial(lambda nd, *_: tuple(0 for _ in range(nd)), a.ndim))
             for a in args]
    return pl.pallas_call(
        lambda *r: _dev_gat_body(*r),
        grid=(1,),
        in_specs=specs,
        out_specs=pl.BlockSpec((_ND, _NH), lambda i: (0, 0)),
        out_shape=jax.ShapeDtypeStruct((_ND, _NH), jnp.float32),
    )(*args)


def _gat_conv(x, src, dst, p, heads, out_dim, use_elu):
    N = x.shape[0]
    z = x @ p['W']
    zr = z.reshape(N, heads, out_dim)
    el = jnp.sum(zr * p['a_l'][None, :, :], axis=-1)
    er = jnp.sum(zr * p['a_r'][None, :, :], axis=-1)
    src_tab = jnp.concatenate([z, el], axis=1)
    g_src = src_tab[src]
    er_d = er[dst]
    e = jax.nn.leaky_relu(g_src[:, heads * out_dim:] + er_d, 0.2)
    ee = jnp.exp(e - jnp.max(e))
    z_src = g_src[:, :heads * out_dim].reshape(-1, heads, out_dim)
    payload = jnp.concatenate(
        [ee, (z_src * ee[:, :, None]).reshape(-1, heads * out_dim)], axis=1)
    s = jax.ops.segment_sum(payload, dst, num_segments=N)
    denom = s[:, :heads]
    num = s[:, heads:].reshape(N, heads, out_dim)
    out = num / (denom[:, :, None] + 1e-9)
    if use_elu:
        out = jax.nn.elu(out)
    return out.reshape(N, heads * out_dim)


def kernel(computation_features, device_features, comp_edge_index, dev_edge_index, params):
    comp_cfg = [(8, True), (8, True), (8, True), (1, False)]
    x = computation_features
    for i, (h, act) in enumerate(comp_cfg):
        x = _gat_conv(x, comp_edge_index[0], comp_edge_index[1],
                      params['comp%d' % i], h, _NH, act)
    comp_emb = x
    dev_emb = _dev_gat(device_features, dev_edge_index, params)
    return _bigru_head(comp_emb, dev_emb, params)


# speedup vs baseline: 11.7147x; 1.0365x over previous
import functools

import jax
import jax.numpy as jnp
from jax.experimental import pallas as pl
from jax.experimental.pallas import tpu as pltpu

_NC = 10000
_ND = 64
_NH = 8
_RNN = 8
_T = 64
_B = 2048
_NC_PAD = 10240


def _bigru_head_body(ce_ref, dbf_ref, dbb_ref, kcf_ref, kcb_ref,
                     rf_ref, rb_ref, brf_ref, brb_ref,
                     wdf_ref, wdb_ref, dbias_ref,
                     out_ref, f_scr, b_scr):
    ce = ce_ref[...]
    af = jnp.dot(kcf_ref[...], ce, preferred_element_type=jnp.float32)
    ab = jnp.dot(kcb_ref[...], ce, preferred_element_type=jnp.float32)
    rf = rf_ref[...]
    rb = rb_ref[...]
    brf = brf_ref[...]
    brb = brb_ref[...]

    def scan_dir(a, db_ref, rT, br, scr_ref, reverse):
        def step(i, h):
            t = (_T - 1 - i) if reverse else i
            d = db_ref[pl.ds(t, 1), :, :]
            mx = a + d.reshape(d.shape[1], d.shape[2])
            mh = jnp.dot(rT, h, preferred_element_type=jnp.float32) + br
            xz, xr, xh = mx[0:8], mx[8:16], mx[16:24]
            hz, hr, hh = mh[0:8], mh[8:16], mh[16:24]
            z = jax.nn.sigmoid(xz + hz)
            r = jax.nn.sigmoid(xr + hr)
            hc = jnp.tanh(xh + r * hh)
            hn = z * h + (1.0 - z) * hc
            scr_ref[pl.ds(t, 1), :, :] = hn.reshape(1, hn.shape[0], hn.shape[1])
            return hn
        h0 = jnp.zeros((_RNN, ce.shape[1]), jnp.float32)
        jax.lax.fori_loop(0, _T, step, h0)

    scan_dir(af, dbf_ref, rf, brf, f_scr, False)
    scan_dir(ab, dbb_ref, rb, brb, b_scr, True)

    wdf = wdf_ref[...]
    wdb = wdb_ref[...]
    dbias = dbias_ref[...]

    def head(t, _):
        f = f_scr[pl.ds(t, 1), :, :]
        b = b_scr[pl.ds(t, 1), :, :]
        f = f.reshape(f.shape[1], f.shape[2])
        b = b.reshape(b.shape[1], b.shape[2])
        lg = (jnp.dot(wdf, f, preferred_element_type=jnp.float32)
              + jnp.dot(wdb, b, preferred_element_type=jnp.float32) + dbias)
        m = jnp.max(lg, axis=0, keepdims=True)
        e = jnp.exp(lg - m)
        s = jnp.sum(e, axis=0, keepdims=True)
        p = e / s
        out_ref[pl.ds(t, 1), :, :] = p.reshape(1, p.shape[0], p.shape[1])
        return 0
    jax.lax.fori_loop(0, _T, head, 0)


def _bigru_head(comp_emb, dev_emb, params):
    pf, pb = params['gru_fwd'], params['gru_bwd']
    ceT = jnp.zeros((_RNN, _NC_PAD), jnp.float32).at[:, :_NC].set(comp_emb.T)

    def dproj(p):
        d = dev_emb @ p['kernel'][_RNN:] + p['bias'][0]
        return jnp.broadcast_to(d[:, :, None], (_T, 3 * _RNN, _B))

    dbf, dbb = dproj(pf), dproj(pb)
    kcf = pf['kernel'][:_RNN].T
    kcb = pb['kernel'][:_RNN].T
    rf, rb = pf['rkernel'].T, pb['rkernel'].T
    brf = pf['bias'][1][:, None]
    brb = pb['bias'][1][:, None]
    wd = params['dense_W'].T
    wdf, wdb = wd[:, :_RNN], wd[:, _RNN:]
    dbias = params['dense_b'][:, None]

    grid = _NC_PAD // _B
    const = lambda shape: pl.BlockSpec(shape, lambda i: tuple(0 for _ in shape))
    out = pl.pallas_call(
        _bigru_head_body,
        grid=(grid,),
        in_specs=[
            pl.BlockSpec((_RNN, _B), lambda i: (0, i)),
            const((_T, 3 * _RNN, _B)),
            const((_T, 3 * _RNN, _B)),
            const((3 * _RNN, _RNN)),
            const((3 * _RNN, _RNN)),
            const((3 * _RNN, _RNN)),
            const((3 * _RNN, _RNN)),
            const((3 * _RNN, 1)),
            const((3 * _RNN, 1)),
            const((3, _RNN)),
            const((3, _RNN)),
            const((3, 1)),
        ],
        out_specs=pl.BlockSpec((_T, 3, _B), lambda i: (0, 0, i)),
        out_shape=jax.ShapeDtypeStruct((_T, 3, _NC_PAD), jnp.float32),
        scratch_shapes=[
            pltpu.VMEM((_T, _RNN, _B), jnp.float32),
            pltpu.VMEM((_T, _RNN, _B), jnp.float32),
        ],
    )(ceT, dbf, dbb, kcf, kcb, rf, rb, brf, brb, wdf, wdb, dbias)
    return out.transpose(2, 0, 1)[:_NC]


def _dev_gat_body(x_ref, src_ref, dst_ref, *refs):
    out_ref = refs[-1]
    wrefs = refs[:-1]
    E, N = 1024, _ND
    src = src_ref[...]
    dst = dst_ref[...]
    iota_e_n = jax.lax.broadcasted_iota(jnp.int32, (E, N), 1)
    s_src = (iota_e_n == src.reshape(E, 1)).astype(jnp.float32)
    s_dst = (iota_e_n == dst.reshape(E, 1)).astype(jnp.float32)
    iota_n_e = jax.lax.broadcasted_iota(jnp.int32, (N, E), 0)
    dmask = (iota_n_e == dst.reshape(1, E))
    s_dstT = dmask.astype(jnp.float32)

    x = x_ref[...]
    layer_cfg = [(8, True), (8, True), (8, True), (1, False)]
    for li, (heads, use_elu) in enumerate(layer_cfg):
        W = wrefs[3 * li][...]
        a_l = wrefs[3 * li + 1][...]
        a_r = wrefs[3 * li + 2][...]
        z = jnp.dot(x, W, preferred_element_type=jnp.float32)
        zr = z.reshape(N, heads, _NH)
        el = jnp.sum(zr * a_l[None, :, :], axis=-1)
        er = jnp.sum(zr * a_r[None, :, :], axis=-1)
        el_s = jnp.dot(s_src, el, preferred_element_type=jnp.float32)
        er_d = jnp.dot(s_dst, er, preferred_element_type=jnp.float32)
        e = el_s + er_d
        e = jnp.where(e >= 0, e, 0.2 * e)
        neg = jnp.float32(-jnp.inf)
        emax_cols = []
        for h in range(heads):
            vh = e[:, h].reshape(1, E)
            mh = jnp.where(dmask, vh, neg)
            emax_cols.append(jnp.max(mh, axis=1, keepdims=True))
        emax = jnp.concatenate(emax_cols, axis=1)
        emax = jnp.where(jnp.isfinite(emax), emax, 0.0)
        emax_d = jnp.dot(s_dst, emax, preferred_element_type=jnp.float32)
        ee = jnp.exp(e - emax_d)
        denom = jnp.dot(s_dstT, ee, preferred_element_type=jnp.float32)
        denom_d = jnp.dot(s_dst, denom, preferred_element_type=jnp.float32)
        alpha = ee / (denom_d + 1e-9)
        z_s = jnp.dot(s_src, z, preferred_element_type=jnp.float32)
        w = z_s.reshape(E, heads, _NH) * alpha[:, :, None]
        w = w.reshape(E, heads * _NH)
        out = jnp.dot(s_dstT, w, preferred_element_type=jnp.float32)
        if use_elu:
            out = jnp.where(out > 0, out, jnp.exp(jnp.minimum(out, 0.0)) - 1.0)
        x = out
    out_ref[...] = x


def _dev_gat(dev_feat, dev_ei, params):
    src = dev_ei[0].reshape(1, -1).astype(jnp.int32)
    dst = dev_ei[1].reshape(1, -1).astype(jnp.int32)
    args = [dev_feat, src, dst]
    for i in range(4):
        p = params['dev%d' % i]
        args += [p['W'], p['a_l'], p['a_r']]
    specs = [pl.BlockSpec(a.shape, functools.partial(lambda nd, *_: tuple(0 for _ in range(nd)), a.ndim))
             for a in args]
    return pl.pallas_call(
        lambda *r: _dev_gat_body(*r),
        grid=(1,),
        in_specs=specs,
        out_specs=pl.BlockSpec((_ND, _NH), lambda i: (0, 0)),
        out_shape=jax.ShapeDtypeStruct((_ND, _NH), jnp.float32),
    )(*args)


def _gat_conv(x, src, dst, p, heads, out_dim, use_elu):
    N = x.shape[0]
    z = x @ p['W']
    zr = z.reshape(N, heads, out_dim)
    er = jnp.sum(zr * p['a_r'][None, :, :], axis=-1)
    z_src = z[src].reshape(-1, heads, out_dim)
    er_d = er[dst]
    el_s = jnp.sum(z_src * p['a_l'][None, :, :], axis=-1)
    e = jax.nn.leaky_relu(el_s + er_d, 0.2)
    ee = jnp.exp(e - jnp.max(e))
    payload = jnp.concatenate(
        [ee, (z_src * ee[:, :, None]).reshape(-1, heads * out_dim)], axis=1)
    s = jax.ops.segment_sum(payload, dst, num_segments=N)
    denom = s[:, :heads]
    num = s[:, heads:].reshape(N, heads, out_dim)
    out = num / (denom[:, :, None] + 1e-9)
    if use_elu:
        out = jax.nn.elu(out)
    return out.reshape(N, heads * out_dim)


def kernel(computation_features, device_features, comp_edge_index, dev_edge_index, params):
    comp_cfg = [(8, True), (8, True), (8, True), (1, False)]
    x = computation_features
    for i, (h, act) in enumerate(comp_cfg):
        x = _gat_conv(x, comp_edge_index[0], comp_edge_index[1],
                      params['comp%d' % i], h, _NH, act)
    comp_emb = x
    dev_emb = _dev_gat(device_features, dev_edge_index, params)
    return _bigru_head(comp_emb, dev_emb, params)
